# Initial kernel scaffold; baseline (speedup 1.0000x reference)
import jax
import jax.numpy as jnp
from jax.experimental import pallas as pl


def kernel(token_ids, table):
    raise NotImplementedError("write your pallas kernel here")


# speedup vs baseline: 4.6627x; 4.6627x over previous
import functools

import jax
import jax.numpy as jnp
from jax import lax
from jax.experimental import pallas as pl
from jax.experimental.pallas import tpu as pltpu
from jax.experimental.pallas import tpu_sc as plsc

_EMB = 64

_NC = 2
_NS = 16
_NW = _NC * _NS

_CHUNK = 800


def _make_gather(batch: int):
  b_per_w = batch // _NW
  n_chunks = b_per_w // _CHUNK
  mesh = plsc.VectorSubcoreMesh(core_axis_name="c", subcore_axis_name="s")

  @functools.partial(
      pl.kernel,
      out_type=jax.ShapeDtypeStruct((batch, _EMB), jnp.float32),
      mesh=mesh,
      compiler_params=pltpu.CompilerParams(use_tc_tiling_on_sc=False),
      scratch_types=[
          pltpu.VMEM((b_per_w,), jnp.int32),
          pltpu.VMEM((2, _CHUNK, _EMB), jnp.float32),
          pltpu.SemaphoreType.DMA,
          pltpu.SemaphoreType.DMA,
          pltpu.SemaphoreType.DMA,
          pltpu.SemaphoreType.DMA,
      ],
  )
  def gather_kernel(table_hbm, idx_hbm, out_hbm, idx_v, rows_v, g0, g1, o0, o1):
    gsem = (g0, g1)
    osem = (o0, o1)
    wid = lax.axis_index("s") * _NC + lax.axis_index("c")
    base = wid * b_per_w
    pltpu.sync_copy(idx_hbm.at[pl.ds(base, b_per_w)], idx_v)

    gathers = [None, None]
    out_copies = [None, None]
    for c in range(n_chunks):
      s = c % 2
      if out_copies[s] is not None:
        out_copies[s].wait()
        out_copies[s] = None
      gathers[s] = pltpu.make_async_copy(
          table_hbm.at[idx_v.at[pl.ds(c * _CHUNK, _CHUNK)]],
          rows_v.at[s],
          gsem[s],
      )
      gathers[s].start()
      if c >= 1:
        sp = (c - 1) % 2
        gathers[sp].wait()
        out_copies[sp] = pltpu.make_async_copy(
            rows_v.at[sp],
            out_hbm.at[pl.ds(base + (c - 1) * _CHUNK, _CHUNK)],
            osem[sp],
        )
        out_copies[sp].start()
    sl = (n_chunks - 1) % 2
    gathers[sl].wait()
    final_copy = pltpu.make_async_copy(
        rows_v.at[sl],
        out_hbm.at[pl.ds(base + (n_chunks - 1) * _CHUNK, _CHUNK)],
        osem[sl],
    )
    final_copy.start()
    for s in (0, 1):
      if out_copies[s] is not None:
        out_copies[s].wait()
    final_copy.wait()

  return gather_kernel


def kernel(token_ids, table):
  batch, seq = token_ids.shape
  flat_ids = token_ids.reshape(-1).astype(jnp.int32)
  out = _make_gather(batch * seq)(table, flat_ids)
  return out.reshape(batch, seq, _EMB)


# speedup vs baseline: 4.6641x; 1.0003x over previous
import functools

import jax
import jax.numpy as jnp
from jax import lax
from jax.experimental import pallas as pl
from jax.experimental.pallas import tpu as pltpu
from jax.experimental.pallas import tpu_sc as plsc

_EMB = 64

_NC = 2
_NS = 16
_NW = _NC * _NS

_CHUNK = 400
_NBUF = 4
_LOOKAHEAD = 3


def _make_gather(batch: int):
  b_per_w = batch // _NW
  n_chunks = b_per_w // _CHUNK
  mesh = plsc.VectorSubcoreMesh(core_axis_name="c", subcore_axis_name="s")

  @functools.partial(
      pl.kernel,
      out_type=jax.ShapeDtypeStruct((batch, _EMB), jnp.float32),
      mesh=mesh,
      compiler_params=pltpu.CompilerParams(use_tc_tiling_on_sc=False),
      scratch_types=[
          pltpu.VMEM((b_per_w,), jnp.int32),
          pltpu.VMEM((_NBUF, _CHUNK, _EMB), jnp.float32),
      ]
      + [pltpu.SemaphoreType.DMA] * (2 * _NBUF),
  )
  def gather_kernel(table_hbm, idx_hbm, out_hbm, idx_v, rows_v, *sems):
    gsem = sems[:_NBUF]
    osem = sems[_NBUF:]
    wid = lax.axis_index("s") * _NC + lax.axis_index("c")
    base = wid * b_per_w
    pltpu.sync_copy(idx_hbm.at[pl.ds(base, b_per_w)], idx_v)

    gathers = [None] * _NBUF
    out_copies = [None] * _NBUF
    for c in range(n_chunks + _LOOKAHEAD):
      if c < n_chunks:
        s = c % _NBUF
        if out_copies[s] is not None:
          out_copies[s].wait()
          out_copies[s] = None
        gathers[s] = pltpu.make_async_copy(
            table_hbm.at[idx_v.at[pl.ds(c * _CHUNK, _CHUNK)]],
            rows_v.at[s],
            gsem[s],
        )
        gathers[s].start()
      d = c - _LOOKAHEAD
      if d >= 0:
        sd = d % _NBUF
        gathers[sd].wait()
        out_copies[sd] = pltpu.make_async_copy(
            rows_v.at[sd],
            out_hbm.at[pl.ds(base + d * _CHUNK, _CHUNK)],
            osem[sd],
        )
        out_copies[sd].start()
    for s in range(_NBUF):
      if out_copies[s] is not None:
        out_copies[s].wait()

  return gather_kernel


def kernel(token_ids, table):
  batch, seq = token_ids.shape
  flat_ids = token_ids.reshape(-1).astype(jnp.int32)
  out = _make_gather(batch * seq)(table, flat_ids)
  return out.reshape(batch, seq, _EMB)


# speedup vs baseline: 7.8882x; 1.6913x over previous
import functools

import jax
import jax.numpy as jnp
from jax import lax
from jax.experimental import pallas as pl
from jax.experimental.pallas import tpu as pltpu
from jax.experimental.pallas import tpu_sc as plsc

_EMB = 64

_NC = 2
_NS = 16
_NW = _NC * _NS

_BL = 128


def _make_lookup(seq: int, batch: int):
  n_bh = batch // _BL
  assert n_bh == _NW
  mesh = plsc.VectorSubcoreMesh(core_axis_name="c", subcore_axis_name="s")

  @functools.partial(
      pl.kernel,
      out_type=jax.ShapeDtypeStruct((seq * 8, n_bh, 8 * _BL), jnp.float32),
      mesh=mesh,
      compiler_params=pltpu.CompilerParams(
          use_tc_tiling_on_sc=False, needs_layout_passes=False),
      scratch_types=[
          pltpu.VMEM((5, _BL), jnp.int32),
          pltpu.VMEM((5, _BL, _EMB), jnp.float32),
          pltpu.VMEM((2, 8 * _BL * 8), jnp.float32),
      ]
      + [pltpu.SemaphoreType.DMA] * 12,
  )
  def body(table_hbm, ids_hbm, out_hbm, idx_v, gbuf, tbuf, *sems):
    isem = sems[0:5]
    gsem = sems[5:10]
    osem = sems[10:12]
    w = lax.axis_index("s") * _NC + lax.axis_index("c")
    base_b = w * _BL

    def idx_copy(s, r):
      return pltpu.make_async_copy(
          ids_hbm.at[s, pl.ds(base_b, _BL)], idx_v.at[r], isem[r])

    def gather_copy(r):
      return pltpu.make_async_copy(
          table_hbm.at[idx_v.at[r]], gbuf.at[r], gsem[r])

    def out_copies(s, r):
      return [
          pltpu.make_async_copy(
              tbuf.at[r, pl.ds(eh * 8 * _BL, 8 * _BL)],
              out_hbm.at[8 * s + eh, w],
              osem[r])
          for eh in range(8)
      ]

    iota16 = lax.iota(jnp.int32, 16)

    for j in range(4):
      idx_copy(j, j).start()
    for j in range(3):
      idx_copy(j, j).wait()
      gather_copy(j).start()
    for s in range(seq):
      r = s % 5
      rt = s % 2
      if s + 4 < seq:
        idx_copy(s + 4, (s + 4) % 5).start()
      if s + 3 < seq:
        idx_copy(s + 3, (s + 3) % 5).wait()
        gather_copy((s + 3) % 5).start()
      gather_copy(r).wait()
      if s >= 2:
        for c in out_copies(s - 2, rt):
          c.wait()

      @plsc.parallel_loop(0, (_EMB // 16) * (_BL // 16) * 16, 1, unroll=4)
      def trans_body(g, r=r, rt=rt):
        eb = (g >> 7) << 4
        bb = ((g >> 4) & 7) << 4
        d = g & 15
        idxb = iota16 + bb
        idxe = jnp.bitwise_and(iota16 + d, 15) + eb
        v = plsc.load_gather(gbuf.at[r], [idxb, idxe])
        sidx = jnp.bitwise_or(jnp.left_shift(idxe, 7), idxb)
        plsc.store_scatter(tbuf.at[rt], [sidx], v)
      for c in out_copies(s, rt):
        c.start()
    for c in out_copies(seq - 2, (seq - 2) % 2):
      c.wait()
    for c in out_copies(seq - 1, (seq - 1) % 2):
      c.wait()

  return body


def kernel(token_ids, table):
  batch, seq = token_ids.shape
  ids_t = jnp.transpose(token_ids).astype(jnp.int32)
  out3 = _make_lookup(seq, batch)(table, ids_t)
  out5 = out3.reshape(seq, _EMB // 8, batch // _BL, 8, _BL)
  return out5.transpose(2, 4, 0, 1, 3).reshape(batch, seq, _EMB)


# speedup vs baseline: 8.0163x; 1.0162x over previous
import functools

import jax
import jax.numpy as jnp
from jax import lax
from jax.experimental import pallas as pl
from jax.experimental.pallas import tpu as pltpu
from jax.experimental.pallas import tpu_sc as plsc

_EMB = 64

_NC = 2
_NS = 16
_NW = _NC * _NS

_BL = 128


def _make_lookup(seq: int, batch: int):
  n_bh = batch // _BL
  assert n_bh == _NW
  mesh = plsc.VectorSubcoreMesh(core_axis_name="c", subcore_axis_name="s")

  @functools.partial(
      pl.kernel,
      out_type=jax.ShapeDtypeStruct((seq * 8, n_bh, 8 * _BL), jnp.float32),
      mesh=mesh,
      compiler_params=pltpu.CompilerParams(
          use_tc_tiling_on_sc=False, needs_layout_passes=False),
      scratch_types=[
          pltpu.VMEM((seq, _BL), jnp.int32),
          pltpu.VMEM((5, _BL, _EMB), jnp.float32),
          pltpu.VMEM((2, 8 * _BL * 8), jnp.float32),
      ]
      + [pltpu.SemaphoreType.DMA] * 7,
  )
  def body(table_hbm, ids_hbm, out_hbm, idx_v, gbuf, tbuf, *sems):
    gsem = sems[0:5]
    osem = sems[5:7]
    w = lax.axis_index("s") * _NC + lax.axis_index("c")
    base_b = w * _BL

    pltpu.sync_copy(ids_hbm.at[:, pl.ds(base_b, _BL)], idx_v)

    def gather_copy(s, r):
      return pltpu.make_async_copy(
          table_hbm.at[idx_v.at[s]], gbuf.at[r], gsem[r])

    def out_copies(s, r):
      return [
          pltpu.make_async_copy(
              tbuf.at[r, pl.ds(eh * 8 * _BL, 8 * _BL)],
              out_hbm.at[8 * s + eh, w],
              osem[r])
          for eh in range(8)
      ]

    iota16 = lax.iota(jnp.int32, 16)

    for j in range(3):
      gather_copy(j, j).start()
    for s in range(seq):
      r = s % 5
      rt = s % 2
      if s + 3 < seq:
        gather_copy(s + 3, (s + 3) % 5).start()
      gather_copy(s, r).wait()
      if s >= 2:
        for c in out_copies(s - 2, rt):
          c.wait()

      @plsc.parallel_loop(0, (_EMB // 16) * (_BL // 16) * 16, 1, unroll=8)
      def trans_body(g, r=r, rt=rt):
        eb = (g >> 7) << 4
        bb = ((g >> 4) & 7) << 4
        d = g & 15
        idxb = iota16 + bb
        idxe = jnp.bitwise_and(iota16 + d, 15) + eb
        v = plsc.load_gather(gbuf.at[r], [idxb, idxe])
        sidx = jnp.bitwise_or(jnp.left_shift(idxe, 7), idxb)
        plsc.store_scatter(tbuf.at[rt], [sidx], v)
      for c in out_copies(s, rt):
        c.start()
    for c in out_copies(seq - 2, (seq - 2) % 2):
      c.wait()
    for c in out_copies(seq - 1, (seq - 1) % 2):
      c.wait()

  return body


def kernel(token_ids, table):
  batch, seq = token_ids.shape
  ids_t = jnp.transpose(token_ids).astype(jnp.int32)
  out3 = _make_lookup(seq, batch)(table, ids_t)
  out5 = out3.reshape(seq, _EMB // 8, batch // _BL, 8, _BL)
  return out5.transpose(2, 4, 0, 1, 3).reshape(batch, seq, _EMB)
